# baseline (speedup 1.0000x reference)
import dataclasses
import functools

import jax
import jax.numpy as jnp
from jax import lax
from jax.experimental import pallas as pl
from jax.experimental.pallas import tpu as pltpu
from jax.experimental.pallas import tpu_sc as plsc

B = 16384
F = 26
E = 16
D0 = F * E
BF = B * F
VOCAB = 1000000
VROWS = VOCAB * E // 128

_NC = 2
_NS = 16
_NW = _NC * _NS
_PER_W = BF // _NW
_CH = 512
_NCHUNK = _PER_W // _CH

_BB = 1024


_SCH = 128
_S_PER_W = B // _NW
_NSCH = _S_PER_W // _SCH
_LCH = _SCH * F
_ICH = 416
_NICH = _LCH // _ICH


def _sc_gather(v_flat, w_flat, idx2d_in):
    mesh = plsc.VectorSubcoreMesh(core_axis_name="c", subcore_axis_name="s")
    cp = pltpu.CompilerParams()
    if "needs_layout_passes" in pltpu.CompilerParams.__dataclass_fields__:
        cp = dataclasses.replace(cp, needs_layout_passes=False)

    @functools.partial(
        pl.kernel,
        mesh=mesh,
        compiler_params=cp,
        out_type=(
            jax.ShapeDtypeStruct((BF * E,), jnp.float32),
            jax.ShapeDtypeStruct((BF,), jnp.float32),
        ),
        scratch_types=[
            pltpu.VMEM((_SCH, F), jnp.int32),
            pltpu.VMEM((_LCH,), jnp.int32),
            pltpu.VMEM((_LCH * E,), jnp.int32),
            pltpu.VMEM((_LCH * E,), jnp.float32),
            pltpu.VMEM((_LCH,), jnp.float32),
        ],
    )
    def k(vt_hbm, wt_hbm, idxt_hbm, ov_hbm, ow_hbm,
          idx2d, idx_v, eidx_v, vbuf, wbuf):
        wid = lax.axis_index("s") * _NC + lax.axis_index("c")
        sbase = wid * _S_PER_W
        iota16 = lax.iota(jnp.int32, 16)
        fidx_lo = iota16
        fidx_hi = iota16 + (F - 16)
        for c in range(_NSCH):
            b0 = sbase + c * _SCH
            o = b0 * F
            pltpu.sync_copy(idxt_hbm.at[pl.ds(b0, _SCH)], idx2d)

            @pl.loop(0, _SCH)
            def _(b):
                b16 = jnp.full((16,), b, jnp.int32)
                lo = plsc.load_gather(idx2d, [b16, fidx_lo])
                hi = plsc.load_gather(idx2d, [b16, fidx_hi])
                idx_v[pl.ds(b * F, 16)] = lo
                idx_v[pl.ds(b * F + (F - 16), 16)] = hi

            @pl.loop(0, _LCH)
            def _(t):
                t16 = jnp.full((16,), t, jnp.int32)
                reg = plsc.load_gather(idx_v, [t16])
                eidx_v[pl.ds(t * E, 16)] = reg * E + iota16

            pltpu.sync_copy(wt_hbm.at[idx_v], wbuf)
            pltpu.sync_copy(wbuf, ow_hbm.at[pl.ds(o, _LCH)])
            pltpu.sync_copy(vt_hbm.at[eidx_v], vbuf)
            pltpu.sync_copy(vbuf, ov_hbm.at[pl.ds(o * E, _LCH * E)])

    return k(v_flat, w_flat, idx2d_in)


def _tc_body(xg_ref, wg_ref, wc_ref, w2_ref, w3_ref, c1_ref, c2_ref, c3_ref,
             o_ref):
    x = xg_ref[...]
    xb = x.astype(jnp.bfloat16)
    acc = lax.dot_general(xb, wc_ref[...], (((1,), (0,)), ((), ())),
                          preferred_element_type=jnp.float32)
    h1 = jnp.maximum(acc[:, :256] + c1_ref[...], 0.0)
    s = acc[:, 256:272]
    sumsq = jnp.sum(x * x, axis=1, keepdims=True)
    fm = 0.5 * (jnp.sum(s * s, axis=1, keepdims=True) - sumsq)
    wsum = jnp.sum(wg_ref[...], axis=1, keepdims=True)
    h2 = jnp.maximum(
        lax.dot_general(h1.astype(jnp.bfloat16), w2_ref[...],
                        (((1,), (0,)), ((), ())),
                        preferred_element_type=jnp.float32) + c2_ref[...], 0.0)
    h3 = jnp.sum(h2 * w3_ref[...], axis=1, keepdims=True)
    o_ref[...] = jax.nn.sigmoid(fm + wsum + h3 + c3_ref[...])


def _tc_call(xg, wgr, wc, w2, w3, c1, c2, c3, interpret=False):
    const = lambda i: (0, 0)
    return pl.pallas_call(
        _tc_body,
        grid=(B // _BB,),
        in_specs=[
            pl.BlockSpec((_BB, D0), lambda i: (i, 0)),
            pl.BlockSpec((_BB, F), lambda i: (i, 0)),
            pl.BlockSpec((D0, 272), const),
            pl.BlockSpec((256, 128), const),
            pl.BlockSpec((1, 128), const),
            pl.BlockSpec((1, 256), const),
            pl.BlockSpec((1, 128), const),
            pl.BlockSpec((1, 1), const),
        ],
        out_specs=pl.BlockSpec((_BB, 1), lambda i: (i, 0)),
        out_shape=jax.ShapeDtypeStruct((B, 1), jnp.float32),
        interpret=interpret,
    )(xg, wgr, wc, w2, w3, c1, c2, c3)


def _fold_weights(W1, b1, W2, b2, W3, b3, w0,
                  bn1a_g, bn1a_b, bn1a_m, bn1a_v, bn1b_g, bn1b_b, bn1b_m,
                  bn1b_v, bn2a_g, bn2a_b, bn2a_m, bn2a_v, bn2b_g, bn2b_b,
                  bn2b_m, bn2b_v):
    def affine(g_a, b_a, m_a, v_a, g_b, b_b, m_b, v_b):
        sa = g_a * lax.rsqrt(v_a + 1e-5)
        ta = b_a - m_a * sa
        sb = g_b * lax.rsqrt(v_b + 1e-5)
        tb = b_b - m_b * sb
        return sa * sb, ta * sb + tb

    s1, t1 = affine(bn1a_g, bn1a_b, bn1a_m, bn1a_v,
                    bn1b_g, bn1b_b, bn1b_m, bn1b_v)
    s2, t2 = affine(bn2a_g, bn2a_b, bn2a_m, bn2a_v,
                    bn2b_g, bn2b_b, bn2b_m, bn2b_v)
    w1f = (W1 * s1[:, None]).T
    ident = jnp.tile(jnp.eye(E, dtype=jnp.float32), (F, 1))
    wc = jnp.concatenate([w1f, ident], axis=1).astype(jnp.bfloat16)
    c1 = (b1 * s1 + t1)[None, :]
    w2f = ((W2 * s2[:, None]).T).astype(jnp.bfloat16)
    c2 = (b2 * s2 + t2)[None, :]
    c3 = (b3 + w0).reshape(1, 1)
    return wc, w2f, W3, c1, c2, c3


def kernel(inputs, w_table, v_table, w0, W1, b1, W2, b2, W3, b3,
           bn1a_g, bn1a_b, bn1a_m, bn1a_v, bn1b_g, bn1b_b, bn1b_m, bn1b_v,
           bn2a_g, bn2a_b, bn2a_m, bn2a_v, bn2b_g, bn2b_b, bn2b_m, bn2b_v):
    vgf, wgf = _sc_gather(v_table.reshape(-1), w_table.reshape(-1), inputs)
    xg = vgf.reshape(B, D0)
    wgr = wgf.reshape(B, F)
    wc, w2f, w3, c1, c2, c3 = _fold_weights(
        W1, b1, W2, b2, W3, b3, w0,
        bn1a_g, bn1a_b, bn1a_m, bn1a_v, bn1b_g, bn1b_b, bn1b_m, bn1b_v,
        bn2a_g, bn2a_b, bn2a_m, bn2a_v, bn2b_g, bn2b_b, bn2b_m, bn2b_v)
    return _tc_call(xg, wgr, wc, w2f, w3, c1, c2, c3)


# speedup vs baseline: 1.0696x; 1.0696x over previous
import dataclasses
import functools

import jax
import jax.numpy as jnp
from jax import lax
from jax.experimental import pallas as pl
from jax.experimental.pallas import tpu as pltpu
from jax.experimental.pallas import tpu_sc as plsc

B = 16384
F = 26
E = 16
D0 = F * E
BF = B * F
VOCAB = 1000000
VROWS = VOCAB * E // 128

_NC = 2
_NS = 16
_NW = _NC * _NS
_PER_W = BF // _NW
_CH = 512
_NCHUNK = _PER_W // _CH

_BB = 1024


_SCH = 128
_S_PER_W = B // _NW
_NSCH = _S_PER_W // _SCH
_LCH = _SCH * F
_ICH = 416
_NICH = _LCH // _ICH


_RSL = 128
_NFULL = VROWS // _RSL
_TAILR = VROWS - _NFULL * _RSL


def _sc_reformat(vt_t, tail128):
    mesh = plsc.VectorSubcoreMesh(core_axis_name="c", subcore_axis_name="s")
    cp = pltpu.CompilerParams()
    if "needs_layout_passes" in pltpu.CompilerParams.__dataclass_fields__:
        cp = dataclasses.replace(cp, needs_layout_passes=False)

    @functools.partial(
        pl.kernel,
        mesh=mesh,
        compiler_params=cp,
        out_type=jax.ShapeDtypeStruct((VROWS, 128), jnp.float32),
        scratch_types=[
            pltpu.VMEM((E, _RSL * 8), jnp.float32),
            pltpu.VMEM((_RSL, 128), jnp.float32),
        ],
    )
    def k(vt_hbm, tail_hbm, o_hbm, slab_in, slab_out):
        wid = lax.axis_index("s") * _NC + lax.axis_index("c")
        iota16 = lax.iota(jnp.int32, 16)

        def do_slab(r0, nrows):
            pltpu.sync_copy(vt_hbm.at[:, pl.ds(r0 * 8, _RSL * 8)], slab_in)

            @pl.loop(0, nrows)
            def _(r):
                r16 = jnp.full((16,), r, jnp.int32)
                for j in range(8):
                    col = jnp.full((16,), r * 8 + j, jnp.int32)
                    reg = plsc.load_gather(slab_in, [iota16, col])
                    plsc.store_scatter(slab_out, [r16, iota16 + j * 16], reg)

            pltpu.sync_copy(slab_out.at[pl.ds(0, nrows)],
                            o_hbm.at[pl.ds(r0, nrows)])

        for t in range(31):
            sl = wid + _NW * t

            @pl.when(sl < _NFULL)
            def _():
                do_slab(sl * _RSL, _RSL)

        @pl.when(wid == _NW - 1)
        def _():
            pltpu.sync_copy(
                vt_hbm.at[:, pl.ds(_NFULL * _RSL * 8, 512)],
                slab_in.at[:, pl.ds(0, 512)])

            @pl.loop(0, 64)
            def _(r):
                r16 = jnp.full((16,), r, jnp.int32)
                for j in range(8):
                    col = jnp.full((16,), r * 8 + j, jnp.int32)
                    reg = plsc.load_gather(slab_in, [iota16, col])
                    plsc.store_scatter(slab_out, [r16, iota16 + j * 16], reg)

            pltpu.sync_copy(slab_out.at[pl.ds(0, 64)],
                            o_hbm.at[pl.ds(_NFULL * _RSL, 64)])

        @pl.when(wid == _NW - 2)
        def _():
            pltpu.sync_copy(tail_hbm, o_hbm.at[pl.ds(VROWS - 8, 8)])

    return k(vt_t, tail128)


def _sc_gather(v128, w_flat, idx2d_in):
    mesh = plsc.VectorSubcoreMesh(core_axis_name="c", subcore_axis_name="s")
    cp = pltpu.CompilerParams()
    if "needs_layout_passes" in pltpu.CompilerParams.__dataclass_fields__:
        cp = dataclasses.replace(cp, needs_layout_passes=False)

    @functools.partial(
        pl.kernel,
        mesh=mesh,
        compiler_params=cp,
        out_type=(
            jax.ShapeDtypeStruct((BF * E,), jnp.float32),
            jax.ShapeDtypeStruct((BF,), jnp.float32),
        ),
        scratch_types=[
            pltpu.VMEM((_SCH, F), jnp.int32),
            pltpu.VMEM((_LCH,), jnp.int32),
            pltpu.VMEM((_LCH,), jnp.int32),
            pltpu.VMEM((_LCH,), jnp.int32),
            pltpu.VMEM((_ICH, 128), jnp.float32),
            pltpu.VMEM((_ICH * E,), jnp.float32),
            pltpu.VMEM((_LCH,), jnp.float32),
        ],
    )
    def k(vt_hbm, wt_hbm, idxt_hbm, ov_hbm, ow_hbm,
          idx2d, idx_v, ridx_v, off_v, rows_v, vbuf, wbuf):
        wid = lax.axis_index("s") * _NC + lax.axis_index("c")
        sbase = wid * _S_PER_W
        iota16 = lax.iota(jnp.int32, 16)
        fidx_lo = iota16
        fidx_hi = iota16 + (F - 16)
        for c in range(_NSCH):
            b0 = sbase + c * _SCH
            o = b0 * F
            pltpu.sync_copy(idxt_hbm.at[pl.ds(b0, _SCH)], idx2d)

            @pl.loop(0, _SCH)
            def _(b):
                b16 = jnp.full((16,), b, jnp.int32)
                lo = plsc.load_gather(idx2d, [b16, fidx_lo])
                hi = plsc.load_gather(idx2d, [b16, fidx_hi])
                idx_v[pl.ds(b * F, 16)] = lo
                idx_v[pl.ds(b * F + (F - 16), 16)] = hi

            @pl.loop(0, _LCH, step=16)
            def _(j):
                reg = idx_v[pl.ds(j, 16)]
                ridx_v[pl.ds(j, 16)] = lax.shift_right_logical(reg, 3)
                off_v[pl.ds(j, 16)] = lax.shift_left(
                    lax.bitwise_and(reg, 7), 4)

            pltpu.sync_copy(wt_hbm.at[idx_v], wbuf)
            pltpu.sync_copy(wbuf, ow_hbm.at[pl.ds(o, _LCH)])

            for ic in range(_NICH):
                go = ic * _ICH
                pltpu.sync_copy(v128_at_rows(vt_hbm, ridx_v, go), rows_v)

                @pl.loop(0, _ICH)
                def _(t):
                    t16 = jnp.full((16,), t, jnp.int32)
                    off = plsc.load_gather(off_v, [t16 + go])
                    vals = plsc.load_gather(rows_v, [t16, off + iota16])
                    vbuf[pl.ds(t * 16, 16)] = vals

                pltpu.sync_copy(
                    vbuf, ov_hbm.at[pl.ds((o + go) * E, _ICH * E)])

    return k(v128, w_flat, idx2d_in)


def v128_at_rows(vt_hbm, ridx_v, go):
    return vt_hbm.at[ridx_v.at[pl.ds(go, _ICH)]]


def _tc_body(xg_ref, wg_ref, wc_ref, w2_ref, w3_ref, c1_ref, c2_ref, c3_ref,
             o_ref):
    x = xg_ref[...]
    xb = x.astype(jnp.bfloat16)
    acc = lax.dot_general(xb, wc_ref[...], (((1,), (0,)), ((), ())),
                          preferred_element_type=jnp.float32)
    h1 = jnp.maximum(acc[:, :256] + c1_ref[...], 0.0)
    s = acc[:, 256:272]
    sumsq = jnp.sum(x * x, axis=1, keepdims=True)
    fm = 0.5 * (jnp.sum(s * s, axis=1, keepdims=True) - sumsq)
    wsum = jnp.sum(wg_ref[...], axis=1, keepdims=True)
    h2 = jnp.maximum(
        lax.dot_general(h1.astype(jnp.bfloat16), w2_ref[...],
                        (((1,), (0,)), ((), ())),
                        preferred_element_type=jnp.float32) + c2_ref[...], 0.0)
    h3 = jnp.sum(h2 * w3_ref[...], axis=1, keepdims=True)
    o_ref[...] = jax.nn.sigmoid(fm + wsum + h3 + c3_ref[...])


def _tc_call(xg, wgr, wc, w2, w3, c1, c2, c3, interpret=False):
    const = lambda i: (0, 0)
    return pl.pallas_call(
        _tc_body,
        grid=(B // _BB,),
        in_specs=[
            pl.BlockSpec((_BB, D0), lambda i: (i, 0)),
            pl.BlockSpec((_BB, F), lambda i: (i, 0)),
            pl.BlockSpec((D0, 272), const),
            pl.BlockSpec((256, 128), const),
            pl.BlockSpec((1, 128), const),
            pl.BlockSpec((1, 256), const),
            pl.BlockSpec((1, 128), const),
            pl.BlockSpec((1, 1), const),
        ],
        out_specs=pl.BlockSpec((_BB, 1), lambda i: (i, 0)),
        out_shape=jax.ShapeDtypeStruct((B, 1), jnp.float32),
        interpret=interpret,
    )(xg, wgr, wc, w2, w3, c1, c2, c3)


def _fold_weights(W1, b1, W2, b2, W3, b3, w0,
                  bn1a_g, bn1a_b, bn1a_m, bn1a_v, bn1b_g, bn1b_b, bn1b_m,
                  bn1b_v, bn2a_g, bn2a_b, bn2a_m, bn2a_v, bn2b_g, bn2b_b,
                  bn2b_m, bn2b_v):
    def affine(g_a, b_a, m_a, v_a, g_b, b_b, m_b, v_b):
        sa = g_a * lax.rsqrt(v_a + 1e-5)
        ta = b_a - m_a * sa
        sb = g_b * lax.rsqrt(v_b + 1e-5)
        tb = b_b - m_b * sb
        return sa * sb, ta * sb + tb

    s1, t1 = affine(bn1a_g, bn1a_b, bn1a_m, bn1a_v,
                    bn1b_g, bn1b_b, bn1b_m, bn1b_v)
    s2, t2 = affine(bn2a_g, bn2a_b, bn2a_m, bn2a_v,
                    bn2b_g, bn2b_b, bn2b_m, bn2b_v)
    w1f = (W1 * s1[:, None]).T
    ident = jnp.tile(jnp.eye(E, dtype=jnp.float32), (F, 1))
    wc = jnp.concatenate([w1f, ident], axis=1).astype(jnp.bfloat16)
    c1 = (b1 * s1 + t1)[None, :]
    w2f = ((W2 * s2[:, None]).T).astype(jnp.bfloat16)
    c2 = (b2 * s2 + t2)[None, :]
    c3 = (b3 + w0).reshape(1, 1)
    return wc, w2f, W3, c1, c2, c3


def kernel(inputs, w_table, v_table, w0, W1, b1, W2, b2, W3, b3,
           bn1a_g, bn1a_b, bn1a_m, bn1a_v, bn1b_g, bn1b_b, bn1b_m, bn1b_v,
           bn2a_g, bn2a_b, bn2a_m, bn2a_v, bn2b_g, bn2b_b, bn2b_m, bn2b_v):
    tail128 = v_table[VOCAB - 64:].reshape(8, 128)
    v128 = _sc_reformat(v_table.T, tail128)
    vgf, wgf = _sc_gather(v128, w_table.reshape(-1), inputs)
    xg = vgf.reshape(B, D0)
    wgr = wgf.reshape(B, F)
    wc, w2f, w3, c1, c2, c3 = _fold_weights(
        W1, b1, W2, b2, W3, b3, w0,
        bn1a_g, bn1a_b, bn1a_m, bn1a_v, bn1b_g, bn1b_b, bn1b_m, bn1b_v,
        bn2a_g, bn2a_b, bn2a_m, bn2a_v, bn2b_g, bn2b_b, bn2b_m, bn2b_v)
    return _tc_call(xg, wgr, wc, w2f, w3, c1, c2, c3)


# speedup vs baseline: 1.5623x; 1.4606x over previous
import dataclasses
import functools

import jax
import jax.numpy as jnp
from jax import lax
from jax.experimental import pallas as pl
from jax.experimental.pallas import tpu as pltpu
from jax.experimental.pallas import tpu_sc as plsc

B = 16384
F = 26
E = 16
D0 = F * E
BF = B * F
VOCAB = 1000000
VROWS = VOCAB * E // 128

_NC = 2
_NS = 16
_NW = _NC * _NS
_PER_W = BF // _NW
_CH = 512
_NCHUNK = _PER_W // _CH

_BB = 1024


_SCH = 128
_S_PER_W = B // _NW
_NSCH = _S_PER_W // _SCH
_LCH = _SCH * F
_ICH = 416
_NICH = _LCH // _ICH


_RSL = 128
_NFULL = VROWS // _RSL
_TAILR = VROWS - _NFULL * _RSL


def _sc_reformat(vt_t, tail128):
    mesh = plsc.VectorSubcoreMesh(core_axis_name="c", subcore_axis_name="s")
    cp = pltpu.CompilerParams()
    if "needs_layout_passes" in pltpu.CompilerParams.__dataclass_fields__:
        cp = dataclasses.replace(cp, needs_layout_passes=False)

    @functools.partial(
        pl.kernel,
        mesh=mesh,
        compiler_params=cp,
        out_type=jax.ShapeDtypeStruct((VROWS, 128), jnp.float32),
        scratch_types=[
            pltpu.VMEM((E, _RSL * 8), jnp.float32),
            pltpu.VMEM((_RSL, 128), jnp.float32),
        ],
    )
    def k(vt_hbm, tail_hbm, o_hbm, slab_in, slab_out):
        wid = lax.axis_index("s") * _NC + lax.axis_index("c")
        iota16 = lax.iota(jnp.int32, 16)

        def do_slab(r0, nrows):
            pltpu.sync_copy(vt_hbm.at[:, pl.ds(r0 * 8, _RSL * 8)], slab_in)

            @pl.loop(0, nrows)
            def _(r):
                r16 = jnp.full((16,), r, jnp.int32)
                regs = [plsc.load_gather(
                            slab_in,
                            [iota16, jnp.full((16,), r * 8 + j, jnp.int32)])
                        for j in range(8)]
                for j in range(8):
                    plsc.store_scatter(
                        slab_out, [r16, iota16 + j * 16], regs[j])

            pltpu.sync_copy(slab_out.at[pl.ds(0, nrows)],
                            o_hbm.at[pl.ds(r0, nrows)])

        for t in range(31):
            sl = wid + _NW * t

            @pl.when(sl < _NFULL)
            def _():
                do_slab(sl * _RSL, _RSL)

        @pl.when(wid == _NW - 1)
        def _():
            pltpu.sync_copy(
                vt_hbm.at[:, pl.ds(_NFULL * _RSL * 8, 512)],
                slab_in.at[:, pl.ds(0, 512)])

            @pl.loop(0, 64)
            def _(r):
                r16 = jnp.full((16,), r, jnp.int32)
                regs = [plsc.load_gather(
                            slab_in,
                            [iota16, jnp.full((16,), r * 8 + j, jnp.int32)])
                        for j in range(8)]
                for j in range(8):
                    plsc.store_scatter(
                        slab_out, [r16, iota16 + j * 16], regs[j])

            pltpu.sync_copy(slab_out.at[pl.ds(0, 64)],
                            o_hbm.at[pl.ds(_NFULL * _RSL, 64)])

        @pl.when(wid == _NW - 2)
        def _():
            pltpu.sync_copy(tail_hbm, o_hbm.at[pl.ds(VROWS - 8, 8)])

    return k(vt_t, tail128)


def _sc_gather(v128, w_flat, idx2d_in):
    mesh = plsc.VectorSubcoreMesh(core_axis_name="c", subcore_axis_name="s")
    cp = pltpu.CompilerParams()
    if "needs_layout_passes" in pltpu.CompilerParams.__dataclass_fields__:
        cp = dataclasses.replace(cp, needs_layout_passes=False)

    @functools.partial(
        pl.kernel,
        mesh=mesh,
        compiler_params=cp,
        out_type=(
            jax.ShapeDtypeStruct((BF * E,), jnp.float32),
            jax.ShapeDtypeStruct((BF,), jnp.float32),
        ),
        scratch_types=[
            pltpu.VMEM((_SCH, F), jnp.int32),
            pltpu.VMEM((_LCH,), jnp.int32),
            pltpu.VMEM((_LCH,), jnp.int32),
            pltpu.VMEM((_LCH,), jnp.int32),
            pltpu.VMEM((_ICH, 128), jnp.float32),
            pltpu.VMEM((_ICH * E,), jnp.float32),
            pltpu.VMEM((_LCH,), jnp.float32),
        ],
    )
    def k(vt_hbm, wt_hbm, idxt_hbm, ov_hbm, ow_hbm,
          idx2d, idx_v, ridx_v, off_v, rows_v, vbuf, wbuf):
        wid = lax.axis_index("s") * _NC + lax.axis_index("c")
        sbase = wid * _S_PER_W
        iota16 = lax.iota(jnp.int32, 16)
        fidx_lo = iota16
        fidx_hi = iota16 + (F - 16)
        for c in range(_NSCH):
            b0 = sbase + c * _SCH
            o = b0 * F
            pltpu.sync_copy(idxt_hbm.at[pl.ds(b0, _SCH)], idx2d)

            @pl.loop(0, _SCH, step=2)
            def _(b):
                b16s = [jnp.full((16,), b + db, jnp.int32) for db in range(2)]
                los = [plsc.load_gather(idx2d, [b16s[db], fidx_lo])
                       for db in range(2)]
                his = [plsc.load_gather(idx2d, [b16s[db], fidx_hi])
                       for db in range(2)]
                for db in range(2):
                    idx_v[pl.ds((b + db) * F, 16)] = los[db]
                    idx_v[pl.ds((b + db) * F + (F - 16), 16)] = his[db]

            @pl.loop(0, _LCH, step=16)
            def _(j):
                reg = idx_v[pl.ds(j, 16)]
                ridx_v[pl.ds(j, 16)] = lax.shift_right_logical(reg, 3)
                off_v[pl.ds(j, 16)] = lax.shift_left(
                    lax.bitwise_and(reg, 7), 4)

            pltpu.sync_copy(wt_hbm.at[idx_v], wbuf)
            pltpu.sync_copy(wbuf, ow_hbm.at[pl.ds(o, _LCH)])

            for ic in range(_NICH):
                go = ic * _ICH
                pltpu.sync_copy(v128_at_rows(vt_hbm, ridx_v, go), rows_v)

                @pl.loop(0, _ICH, step=4)
                def _(t):
                    t16s = [jnp.full((16,), t + dt, jnp.int32)
                            for dt in range(4)]
                    offs = [plsc.load_gather(off_v, [t16s[dt] + go])
                            for dt in range(4)]
                    vals = [plsc.load_gather(
                                rows_v, [t16s[dt], offs[dt] + iota16])
                            for dt in range(4)]
                    for dt in range(4):
                        vbuf[pl.ds((t + dt) * 16, 16)] = vals[dt]

                pltpu.sync_copy(
                    vbuf, ov_hbm.at[pl.ds((o + go) * E, _ICH * E)])

    return k(v128, w_flat, idx2d_in)


def v128_at_rows(vt_hbm, ridx_v, go):
    return vt_hbm.at[ridx_v.at[pl.ds(go, _ICH)]]


def _tc_body(xg_ref, wg_ref, wc_ref, w2_ref, w3_ref, c1_ref, c2_ref, c3_ref,
             o_ref):
    x = xg_ref[...]
    xb = x.astype(jnp.bfloat16)
    acc = lax.dot_general(xb, wc_ref[...], (((1,), (0,)), ((), ())),
                          preferred_element_type=jnp.float32)
    h1 = jnp.maximum(acc[:, :256] + c1_ref[...], 0.0)
    s = acc[:, 256:272]
    sumsq = jnp.sum(x * x, axis=1, keepdims=True)
    fm = 0.5 * (jnp.sum(s * s, axis=1, keepdims=True) - sumsq)
    wsum = jnp.sum(wg_ref[...], axis=1, keepdims=True)
    h2 = jnp.maximum(
        lax.dot_general(h1.astype(jnp.bfloat16), w2_ref[...],
                        (((1,), (0,)), ((), ())),
                        preferred_element_type=jnp.float32) + c2_ref[...], 0.0)
    h3 = jnp.sum(h2 * w3_ref[...], axis=1, keepdims=True)
    o_ref[...] = jax.nn.sigmoid(fm + wsum + h3 + c3_ref[...])


def _tc_call(xg, wgr, wc, w2, w3, c1, c2, c3, interpret=False):
    const = lambda i: (0, 0)
    return pl.pallas_call(
        _tc_body,
        grid=(B // _BB,),
        in_specs=[
            pl.BlockSpec((_BB, D0), lambda i: (i, 0)),
            pl.BlockSpec((_BB, F), lambda i: (i, 0)),
            pl.BlockSpec((D0, 272), const),
            pl.BlockSpec((256, 128), const),
            pl.BlockSpec((1, 128), const),
            pl.BlockSpec((1, 256), const),
            pl.BlockSpec((1, 128), const),
            pl.BlockSpec((1, 1), const),
        ],
        out_specs=pl.BlockSpec((_BB, 1), lambda i: (i, 0)),
        out_shape=jax.ShapeDtypeStruct((B, 1), jnp.float32),
        interpret=interpret,
    )(xg, wgr, wc, w2, w3, c1, c2, c3)


def _fold_weights(W1, b1, W2, b2, W3, b3, w0,
                  bn1a_g, bn1a_b, bn1a_m, bn1a_v, bn1b_g, bn1b_b, bn1b_m,
                  bn1b_v, bn2a_g, bn2a_b, bn2a_m, bn2a_v, bn2b_g, bn2b_b,
                  bn2b_m, bn2b_v):
    def affine(g_a, b_a, m_a, v_a, g_b, b_b, m_b, v_b):
        sa = g_a * lax.rsqrt(v_a + 1e-5)
        ta = b_a - m_a * sa
        sb = g_b * lax.rsqrt(v_b + 1e-5)
        tb = b_b - m_b * sb
        return sa * sb, ta * sb + tb

    s1, t1 = affine(bn1a_g, bn1a_b, bn1a_m, bn1a_v,
                    bn1b_g, bn1b_b, bn1b_m, bn1b_v)
    s2, t2 = affine(bn2a_g, bn2a_b, bn2a_m, bn2a_v,
                    bn2b_g, bn2b_b, bn2b_m, bn2b_v)
    w1f = (W1 * s1[:, None]).T
    ident = jnp.tile(jnp.eye(E, dtype=jnp.float32), (F, 1))
    wc = jnp.concatenate([w1f, ident], axis=1).astype(jnp.bfloat16)
    c1 = (b1 * s1 + t1)[None, :]
    w2f = ((W2 * s2[:, None]).T).astype(jnp.bfloat16)
    c2 = (b2 * s2 + t2)[None, :]
    c3 = (b3 + w0).reshape(1, 1)
    return wc, w2f, W3, c1, c2, c3


def kernel(inputs, w_table, v_table, w0, W1, b1, W2, b2, W3, b3,
           bn1a_g, bn1a_b, bn1a_m, bn1a_v, bn1b_g, bn1b_b, bn1b_m, bn1b_v,
           bn2a_g, bn2a_b, bn2a_m, bn2a_v, bn2b_g, bn2b_b, bn2b_m, bn2b_v):
    tail128 = v_table[VOCAB - 64:].reshape(8, 128)
    v128 = _sc_reformat(v_table.T, tail128)
    vgf, wgf = _sc_gather(v128, w_table.reshape(-1), inputs)
    xg = vgf.reshape(B, D0)
    wgr = wgf.reshape(B, F)
    wc, w2f, w3, c1, c2, c3 = _fold_weights(
        W1, b1, W2, b2, W3, b3, w0,
        bn1a_g, bn1a_b, bn1a_m, bn1a_v, bn1b_g, bn1b_b, bn1b_m, bn1b_v,
        bn2a_g, bn2a_b, bn2a_m, bn2a_v, bn2b_g, bn2b_b, bn2b_m, bn2b_v)
    return _tc_call(xg, wgr, wc, w2f, w3, c1, c2, c3)


# speedup vs baseline: 1.5959x; 1.0215x over previous
import dataclasses
import functools

import jax
import jax.numpy as jnp
from jax import lax
from jax.experimental import pallas as pl
from jax.experimental.pallas import tpu as pltpu
from jax.experimental.pallas import tpu_sc as plsc

B = 16384
F = 26
E = 16
D0 = F * E
BF = B * F
VOCAB = 1000000
VROWS = VOCAB * E // 128

_NC = 2
_NS = 16
_NW = _NC * _NS
_PER_W = BF // _NW
_CH = 512
_NCHUNK = _PER_W // _CH

_BB = 1024


_SCH = 128
_S_PER_W = B // _NW
_NSCH = _S_PER_W // _SCH
_LCH = _SCH * F
_ICH = 416
_NICH = _LCH // _ICH


_RSL = 128
_NFULL = VROWS // _RSL
_TAILR = VROWS - _NFULL * _RSL


def _sc_reformat(vt_t, tail128):
    mesh = plsc.VectorSubcoreMesh(core_axis_name="c", subcore_axis_name="s")
    cp = pltpu.CompilerParams()
    if "needs_layout_passes" in pltpu.CompilerParams.__dataclass_fields__:
        cp = dataclasses.replace(cp, needs_layout_passes=False)

    @functools.partial(
        pl.kernel,
        mesh=mesh,
        compiler_params=cp,
        out_type=jax.ShapeDtypeStruct((VROWS, 128), jnp.float32),
        scratch_types=[
            pltpu.VMEM((E, _RSL * 8), jnp.float32),
            pltpu.VMEM((_RSL, 128), jnp.float32),
        ],
    )
    def k(vt_hbm, tail_hbm, o_hbm, slab_in, slab_out):
        wid = lax.axis_index("s") * _NC + lax.axis_index("c")
        iota16 = lax.iota(jnp.int32, 16)

        def do_slab(r0, nrows):
            pltpu.sync_copy(vt_hbm.at[:, pl.ds(r0 * 8, _RSL * 8)], slab_in)

            @pl.loop(0, nrows)
            def _(r):
                r16 = jnp.full((16,), r, jnp.int32)
                regs = [plsc.load_gather(
                            slab_in,
                            [iota16, jnp.full((16,), r * 8 + j, jnp.int32)])
                        for j in range(8)]
                for j in range(8):
                    plsc.store_scatter(
                        slab_out, [r16, iota16 + j * 16], regs[j])

            pltpu.sync_copy(slab_out.at[pl.ds(0, nrows)],
                            o_hbm.at[pl.ds(r0, nrows)])

        for t in range(31):
            sl = wid + _NW * t

            @pl.when(sl < _NFULL)
            def _():
                do_slab(sl * _RSL, _RSL)

        @pl.when(wid == _NW - 1)
        def _():
            pltpu.sync_copy(
                vt_hbm.at[:, pl.ds(_NFULL * _RSL * 8, 512)],
                slab_in.at[:, pl.ds(0, 512)])

            @pl.loop(0, 64)
            def _(r):
                r16 = jnp.full((16,), r, jnp.int32)
                regs = [plsc.load_gather(
                            slab_in,
                            [iota16, jnp.full((16,), r * 8 + j, jnp.int32)])
                        for j in range(8)]
                for j in range(8):
                    plsc.store_scatter(
                        slab_out, [r16, iota16 + j * 16], regs[j])

            pltpu.sync_copy(slab_out.at[pl.ds(0, 64)],
                            o_hbm.at[pl.ds(_NFULL * _RSL, 64)])

        @pl.when(wid == _NW - 2)
        def _():
            pltpu.sync_copy(tail_hbm, o_hbm.at[pl.ds(VROWS - 8, 8)])

    return k(vt_t, tail128)


def _sc_gather(v128, w_flat, idx2d_in):
    mesh = plsc.VectorSubcoreMesh(core_axis_name="c", subcore_axis_name="s")
    cp = pltpu.CompilerParams()
    if "needs_layout_passes" in pltpu.CompilerParams.__dataclass_fields__:
        cp = dataclasses.replace(cp, needs_layout_passes=False)

    @functools.partial(
        pl.kernel,
        mesh=mesh,
        compiler_params=cp,
        out_type=(
            jax.ShapeDtypeStruct((BF * E,), jnp.float32),
            jax.ShapeDtypeStruct((BF,), jnp.float32),
        ),
        scratch_types=[
            pltpu.VMEM((_SCH, F), jnp.int32),
            pltpu.VMEM((_LCH,), jnp.int32),
            pltpu.VMEM((_LCH,), jnp.int32),
            pltpu.VMEM((_LCH,), jnp.int32),
            pltpu.VMEM((_ICH, 128), jnp.float32),
            pltpu.VMEM((_ICH * E,), jnp.float32),
            pltpu.VMEM((_LCH,), jnp.float32),
        ],
    )
    def k(vt_hbm, wt_hbm, idxt_hbm, ov_hbm, ow_hbm,
          idx2d, idx_v, ridx_v, off_v, rows_v, vbuf, wbuf):
        wid = lax.axis_index("s") * _NC + lax.axis_index("c")
        sbase = wid * _S_PER_W
        iota16 = lax.iota(jnp.int32, 16)
        fidx_lo = iota16
        fidx_hi = iota16 + (F - 16)
        for c in range(_NSCH):
            b0 = sbase + c * _SCH
            o = b0 * F
            pltpu.sync_copy(idxt_hbm.at[pl.ds(b0, _SCH)], idx2d)

            @pl.loop(0, _SCH, step=2)
            def _(b):
                b16s = [jnp.full((16,), b + db, jnp.int32) for db in range(2)]
                los = [plsc.load_gather(idx2d, [b16s[db], fidx_lo])
                       for db in range(2)]
                his = [plsc.load_gather(idx2d, [b16s[db], fidx_hi])
                       for db in range(2)]
                for db in range(2):
                    idx_v[pl.ds((b + db) * F, 16)] = los[db]
                    idx_v[pl.ds((b + db) * F + (F - 16), 16)] = his[db]

            @pl.loop(0, _LCH, step=16)
            def _(j):
                reg = idx_v[pl.ds(j, 16)]
                ridx_v[pl.ds(j, 16)] = lax.shift_right_logical(reg, 3)
                off_v[pl.ds(j, 16)] = lax.shift_left(
                    lax.bitwise_and(reg, 7), 4)

            pltpu.sync_copy(wt_hbm.at[idx_v], wbuf)
            pltpu.sync_copy(wbuf, ow_hbm.at[pl.ds(o, _LCH)])

            for ic in range(_NICH):
                go = ic * _ICH
                pltpu.sync_copy(v128_at_rows(vt_hbm, ridx_v, go), rows_v)

                @pl.loop(0, _ICH, step=8)
                def _(t):
                    t16s = [jnp.full((16,), t + dt, jnp.int32)
                            for dt in range(8)]
                    offs = [plsc.load_gather(off_v, [t16s[dt] + go])
                            for dt in range(8)]
                    vals = [plsc.load_gather(
                                rows_v, [t16s[dt], offs[dt] + iota16])
                            for dt in range(8)]
                    for dt in range(8):
                        vbuf[pl.ds((t + dt) * 16, 16)] = vals[dt]

                pltpu.sync_copy(
                    vbuf, ov_hbm.at[pl.ds((o + go) * E, _ICH * E)])

    return k(v128, w_flat, idx2d_in)


def v128_at_rows(vt_hbm, ridx_v, go):
    return vt_hbm.at[ridx_v.at[pl.ds(go, _ICH)]]


def _tc_body(xg_ref, wg_ref, wc_ref, w2_ref, w3_ref, c1_ref, c2_ref, c3_ref,
             o_ref):
    x = xg_ref[...]
    xb = x.astype(jnp.bfloat16)
    acc = lax.dot_general(xb, wc_ref[...], (((1,), (0,)), ((), ())),
                          preferred_element_type=jnp.float32)
    h1 = jnp.maximum(acc[:, :256] + c1_ref[...], 0.0)
    s = acc[:, 256:272]
    sumsq = jnp.sum(x * x, axis=1, keepdims=True)
    fm = 0.5 * (jnp.sum(s * s, axis=1, keepdims=True) - sumsq)
    wsum = jnp.sum(wg_ref[...], axis=1, keepdims=True)
    h2 = jnp.maximum(
        lax.dot_general(h1.astype(jnp.bfloat16), w2_ref[...],
                        (((1,), (0,)), ((), ())),
                        preferred_element_type=jnp.float32) + c2_ref[...], 0.0)
    h3 = jnp.sum(h2 * w3_ref[...], axis=1, keepdims=True)
    o_ref[...] = jax.nn.sigmoid(fm + wsum + h3 + c3_ref[...])


def _tc_call(xg, wgr, wc, w2, w3, c1, c2, c3, interpret=False):
    const = lambda i: (0, 0)
    return pl.pallas_call(
        _tc_body,
        grid=(B // _BB,),
        in_specs=[
            pl.BlockSpec((_BB, D0), lambda i: (i, 0)),
            pl.BlockSpec((_BB, F), lambda i: (i, 0)),
            pl.BlockSpec((D0, 272), const),
            pl.BlockSpec((256, 128), const),
            pl.BlockSpec((1, 128), const),
            pl.BlockSpec((1, 256), const),
            pl.BlockSpec((1, 128), const),
            pl.BlockSpec((1, 1), const),
        ],
        out_specs=pl.BlockSpec((_BB, 1), lambda i: (i, 0)),
        out_shape=jax.ShapeDtypeStruct((B, 1), jnp.float32),
        interpret=interpret,
    )(xg, wgr, wc, w2, w3, c1, c2, c3)


def _fold_weights(W1, b1, W2, b2, W3, b3, w0,
                  bn1a_g, bn1a_b, bn1a_m, bn1a_v, bn1b_g, bn1b_b, bn1b_m,
                  bn1b_v, bn2a_g, bn2a_b, bn2a_m, bn2a_v, bn2b_g, bn2b_b,
                  bn2b_m, bn2b_v):
    def affine(g_a, b_a, m_a, v_a, g_b, b_b, m_b, v_b):
        sa = g_a * lax.rsqrt(v_a + 1e-5)
        ta = b_a - m_a * sa
        sb = g_b * lax.rsqrt(v_b + 1e-5)
        tb = b_b - m_b * sb
        return sa * sb, ta * sb + tb

    s1, t1 = affine(bn1a_g, bn1a_b, bn1a_m, bn1a_v,
                    bn1b_g, bn1b_b, bn1b_m, bn1b_v)
    s2, t2 = affine(bn2a_g, bn2a_b, bn2a_m, bn2a_v,
                    bn2b_g, bn2b_b, bn2b_m, bn2b_v)
    w1f = (W1 * s1[:, None]).T
    ident = jnp.tile(jnp.eye(E, dtype=jnp.float32), (F, 1))
    wc = jnp.concatenate([w1f, ident], axis=1).astype(jnp.bfloat16)
    c1 = (b1 * s1 + t1)[None, :]
    w2f = ((W2 * s2[:, None]).T).astype(jnp.bfloat16)
    c2 = (b2 * s2 + t2)[None, :]
    c3 = (b3 + w0).reshape(1, 1)
    return wc, w2f, W3, c1, c2, c3


def kernel(inputs, w_table, v_table, w0, W1, b1, W2, b2, W3, b3,
           bn1a_g, bn1a_b, bn1a_m, bn1a_v, bn1b_g, bn1b_b, bn1b_m, bn1b_v,
           bn2a_g, bn2a_b, bn2a_m, bn2a_v, bn2b_g, bn2b_b, bn2b_m, bn2b_v):
    tail128 = v_table[VOCAB - 64:].reshape(8, 128)
    v128 = _sc_reformat(v_table.T, tail128)
    vgf, wgf = _sc_gather(v128, w_table.reshape(-1), inputs)
    xg = vgf.reshape(B, D0)
    wgr = wgf.reshape(B, F)
    wc, w2f, w3, c1, c2, c3 = _fold_weights(
        W1, b1, W2, b2, W3, b3, w0,
        bn1a_g, bn1a_b, bn1a_m, bn1a_v, bn1b_g, bn1b_b, bn1b_m, bn1b_v,
        bn2a_g, bn2a_b, bn2a_m, bn2a_v, bn2b_g, bn2b_b, bn2b_m, bn2b_v)
    return _tc_call(xg, wgr, wc, w2f, w3, c1, c2, c3)


# speedup vs baseline: 1.6089x; 1.0082x over previous
import dataclasses
import functools

import jax
import jax.numpy as jnp
from jax import lax
from jax.experimental import pallas as pl
from jax.experimental.pallas import tpu as pltpu
from jax.experimental.pallas import tpu_sc as plsc

B = 16384
F = 26
E = 16
D0 = F * E
BF = B * F
VOCAB = 1000000
VROWS = VOCAB * E // 128

_NC = 2
_NS = 16
_NW = _NC * _NS
_PER_W = BF // _NW
_CH = 512
_NCHUNK = _PER_W // _CH

_BB = 1024


_SCH = 128
_S_PER_W = B // _NW
_NSCH = _S_PER_W // _SCH
_LCH = _SCH * F
_ICH = 416
_NICH = _LCH // _ICH


_RSL = 128
_NFULL = VROWS // _RSL
_TAILR = VROWS - _NFULL * _RSL


def _sc_reformat(vt_t, tail128):
    mesh = plsc.VectorSubcoreMesh(core_axis_name="c", subcore_axis_name="s")
    cp = pltpu.CompilerParams()
    if "needs_layout_passes" in pltpu.CompilerParams.__dataclass_fields__:
        cp = dataclasses.replace(cp, needs_layout_passes=False)

    @functools.partial(
        pl.kernel,
        mesh=mesh,
        compiler_params=cp,
        out_type=jax.ShapeDtypeStruct((VROWS, 128), jnp.float32),
        scratch_types=[
            pltpu.VMEM((E, _RSL * 8), jnp.float32),
            pltpu.VMEM((_RSL, 128), jnp.float32),
        ],
    )
    def k(vt_hbm, tail_hbm, o_hbm, slab_in, slab_out):
        wid = lax.axis_index("s") * _NC + lax.axis_index("c")
        iota16 = lax.iota(jnp.int32, 16)

        def do_slab(r0, nrows):
            pltpu.sync_copy(vt_hbm.at[:, pl.ds(r0 * 8, _RSL * 8)], slab_in)

            @pl.loop(0, nrows, step=2)
            def _(r):
                r16s = [jnp.full((16,), r + dr, jnp.int32) for dr in range(2)]
                regs = [plsc.load_gather(
                            slab_in,
                            [iota16,
                             jnp.full((16,), (r + dr) * 8 + j, jnp.int32)])
                        for dr in range(2) for j in range(8)]
                for dr in range(2):
                    for j in range(8):
                        plsc.store_scatter(
                            slab_out, [r16s[dr], iota16 + j * 16],
                            regs[dr * 8 + j])

            pltpu.sync_copy(slab_out.at[pl.ds(0, nrows)],
                            o_hbm.at[pl.ds(r0, nrows)])

        for t in range(31):
            sl = wid + _NW * t

            @pl.when(sl < _NFULL)
            def _():
                do_slab(sl * _RSL, _RSL)

        @pl.when(wid == _NW - 1)
        def _():
            pltpu.sync_copy(
                vt_hbm.at[:, pl.ds(_NFULL * _RSL * 8, 512)],
                slab_in.at[:, pl.ds(0, 512)])

            @pl.loop(0, 64)
            def _(r):
                r16 = jnp.full((16,), r, jnp.int32)
                regs = [plsc.load_gather(
                            slab_in,
                            [iota16, jnp.full((16,), r * 8 + j, jnp.int32)])
                        for j in range(8)]
                for j in range(8):
                    plsc.store_scatter(
                        slab_out, [r16, iota16 + j * 16], regs[j])

            pltpu.sync_copy(slab_out.at[pl.ds(0, 64)],
                            o_hbm.at[pl.ds(_NFULL * _RSL, 64)])

        @pl.when(wid == _NW - 2)
        def _():
            pltpu.sync_copy(tail_hbm, o_hbm.at[pl.ds(VROWS - 8, 8)])

    return k(vt_t, tail128)


def _sc_gather(v128, w_flat, idx2d_in):
    mesh = plsc.VectorSubcoreMesh(core_axis_name="c", subcore_axis_name="s")
    cp = pltpu.CompilerParams()
    if "needs_layout_passes" in pltpu.CompilerParams.__dataclass_fields__:
        cp = dataclasses.replace(cp, needs_layout_passes=False)

    @functools.partial(
        pl.kernel,
        mesh=mesh,
        compiler_params=cp,
        out_type=(
            jax.ShapeDtypeStruct((BF * E,), jnp.float32),
            jax.ShapeDtypeStruct((BF,), jnp.float32),
        ),
        scratch_types=[
            pltpu.VMEM((_SCH, F), jnp.int32),
            pltpu.VMEM((_LCH,), jnp.int32),
            pltpu.VMEM((_LCH,), jnp.int32),
            pltpu.VMEM((_LCH,), jnp.int32),
            pltpu.VMEM((_ICH, 128), jnp.float32),
            pltpu.VMEM((_ICH * E,), jnp.float32),
            pltpu.VMEM((_LCH,), jnp.float32),
        ],
    )
    def k(vt_hbm, wt_hbm, idxt_hbm, ov_hbm, ow_hbm,
          idx2d, idx_v, ridx_v, off_v, rows_v, vbuf, wbuf):
        wid = lax.axis_index("s") * _NC + lax.axis_index("c")
        sbase = wid * _S_PER_W
        iota16 = lax.iota(jnp.int32, 16)
        fidx_lo = iota16
        fidx_hi = iota16 + (F - 16)
        for c in range(_NSCH):
            b0 = sbase + c * _SCH
            o = b0 * F
            pltpu.sync_copy(idxt_hbm.at[pl.ds(b0, _SCH)], idx2d)

            @pl.loop(0, _SCH, step=2)
            def _(b):
                b16s = [jnp.full((16,), b + db, jnp.int32) for db in range(2)]
                los = [plsc.load_gather(idx2d, [b16s[db], fidx_lo])
                       for db in range(2)]
                his = [plsc.load_gather(idx2d, [b16s[db], fidx_hi])
                       for db in range(2)]
                for db in range(2):
                    idx_v[pl.ds((b + db) * F, 16)] = los[db]
                    idx_v[pl.ds((b + db) * F + (F - 16), 16)] = his[db]

            @pl.loop(0, _LCH, step=16)
            def _(j):
                reg = idx_v[pl.ds(j, 16)]
                ridx_v[pl.ds(j, 16)] = lax.shift_right_logical(reg, 3)
                off_v[pl.ds(j, 16)] = lax.shift_left(
                    lax.bitwise_and(reg, 7), 4)

            pltpu.sync_copy(wt_hbm.at[idx_v], wbuf)
            pltpu.sync_copy(wbuf, ow_hbm.at[pl.ds(o, _LCH)])

            for ic in range(_NICH):
                go = ic * _ICH
                pltpu.sync_copy(v128_at_rows(vt_hbm, ridx_v, go), rows_v)

                @pl.loop(0, _ICH, step=8)
                def _(t):
                    t16s = [jnp.full((16,), t + dt, jnp.int32)
                            for dt in range(8)]
                    offs = [plsc.load_gather(off_v, [t16s[dt] + go])
                            for dt in range(8)]
                    vals = [plsc.load_gather(
                                rows_v, [t16s[dt], offs[dt] + iota16])
                            for dt in range(8)]
                    for dt in range(8):
                        vbuf[pl.ds((t + dt) * 16, 16)] = vals[dt]

                pltpu.sync_copy(
                    vbuf, ov_hbm.at[pl.ds((o + go) * E, _ICH * E)])

    return k(v128, w_flat, idx2d_in)


def v128_at_rows(vt_hbm, ridx_v, go):
    return vt_hbm.at[ridx_v.at[pl.ds(go, _ICH)]]


def _tc_body(xg_ref, wg_ref, wc_ref, w2_ref, w3_ref, c1_ref, c2_ref, c3_ref,
             o_ref):
    x = xg_ref[...]
    xb = x.astype(jnp.bfloat16)
    acc = lax.dot_general(xb, wc_ref[...], (((1,), (0,)), ((), ())),
                          preferred_element_type=jnp.float32)
    h1 = jnp.maximum(acc[:, :256] + c1_ref[...], 0.0)
    s = acc[:, 256:272]
    sumsq = jnp.sum(x * x, axis=1, keepdims=True)
    fm = 0.5 * (jnp.sum(s * s, axis=1, keepdims=True) - sumsq)
    wsum = jnp.sum(wg_ref[...], axis=1, keepdims=True)
    h2 = jnp.maximum(
        lax.dot_general(h1.astype(jnp.bfloat16), w2_ref[...],
                        (((1,), (0,)), ((), ())),
                        preferred_element_type=jnp.float32) + c2_ref[...], 0.0)
    h3 = jnp.sum(h2 * w3_ref[...], axis=1, keepdims=True)
    o_ref[...] = jax.nn.sigmoid(fm + wsum + h3 + c3_ref[...])


def _tc_call(xg, wgr, wc, w2, w3, c1, c2, c3, interpret=False):
    const = lambda i: (0, 0)
    return pl.pallas_call(
        _tc_body,
        grid=(B // _BB,),
        in_specs=[
            pl.BlockSpec((_BB, D0), lambda i: (i, 0)),
            pl.BlockSpec((_BB, F), lambda i: (i, 0)),
            pl.BlockSpec((D0, 272), const),
            pl.BlockSpec((256, 128), const),
            pl.BlockSpec((1, 128), const),
            pl.BlockSpec((1, 256), const),
            pl.BlockSpec((1, 128), const),
            pl.BlockSpec((1, 1), const),
        ],
        out_specs=pl.BlockSpec((_BB, 1), lambda i: (i, 0)),
        out_shape=jax.ShapeDtypeStruct((B, 1), jnp.float32),
        interpret=interpret,
    )(xg, wgr, wc, w2, w3, c1, c2, c3)


def _fold_weights(W1, b1, W2, b2, W3, b3, w0,
                  bn1a_g, bn1a_b, bn1a_m, bn1a_v, bn1b_g, bn1b_b, bn1b_m,
                  bn1b_v, bn2a_g, bn2a_b, bn2a_m, bn2a_v, bn2b_g, bn2b_b,
                  bn2b_m, bn2b_v):
    def affine(g_a, b_a, m_a, v_a, g_b, b_b, m_b, v_b):
        sa = g_a * lax.rsqrt(v_a + 1e-5)
        ta = b_a - m_a * sa
        sb = g_b * lax.rsqrt(v_b + 1e-5)
        tb = b_b - m_b * sb
        return sa * sb, ta * sb + tb

    s1, t1 = affine(bn1a_g, bn1a_b, bn1a_m, bn1a_v,
                    bn1b_g, bn1b_b, bn1b_m, bn1b_v)
    s2, t2 = affine(bn2a_g, bn2a_b, bn2a_m, bn2a_v,
                    bn2b_g, bn2b_b, bn2b_m, bn2b_v)
    w1f = (W1 * s1[:, None]).T
    ident = jnp.tile(jnp.eye(E, dtype=jnp.float32), (F, 1))
    wc = jnp.concatenate([w1f, ident], axis=1).astype(jnp.bfloat16)
    c1 = (b1 * s1 + t1)[None, :]
    w2f = ((W2 * s2[:, None]).T).astype(jnp.bfloat16)
    c2 = (b2 * s2 + t2)[None, :]
    c3 = (b3 + w0).reshape(1, 1)
    return wc, w2f, W3, c1, c2, c3


def kernel(inputs, w_table, v_table, w0, W1, b1, W2, b2, W3, b3,
           bn1a_g, bn1a_b, bn1a_m, bn1a_v, bn1b_g, bn1b_b, bn1b_m, bn1b_v,
           bn2a_g, bn2a_b, bn2a_m, bn2a_v, bn2b_g, bn2b_b, bn2b_m, bn2b_v):
    tail128 = v_table[VOCAB - 64:].reshape(8, 128)
    v128 = _sc_reformat(v_table.T, tail128)
    vgf, wgf = _sc_gather(v128, w_table.reshape(-1), inputs)
    xg = vgf.reshape(B, D0)
    wgr = wgf.reshape(B, F)
    wc, w2f, w3, c1, c2, c3 = _fold_weights(
        W1, b1, W2, b2, W3, b3, w0,
        bn1a_g, bn1a_b, bn1a_m, bn1a_v, bn1b_g, bn1b_b, bn1b_m, bn1b_v,
        bn2a_g, bn2a_b, bn2a_m, bn2a_v, bn2b_g, bn2b_b, bn2b_m, bn2b_v)
    return _tc_call(xg, wgr, wc, w2f, w3, c1, c2, c3)
